# baseline (speedup 1.0000x reference)
import functools

import jax
import jax.numpy as jnp
from jax import lax
from jax.experimental import pallas as pl
from jax.experimental.pallas import tpu as pltpu
from jax.experimental.pallas import tpu_sc as plsc

N = 10000
E = 320000
D = 128
NUM_ATOM = 119
NUM_BOND = 8
NUM_DIST = 64

NC = 2
NS = 16
L = 16
NW = NC * NS

EPW = E // NW
DEG_ROWS = E // 128
_ROWS_PER_TILE = 160
DST2D_ROWS = _ROWS_PER_TILE * NS


@functools.cache
def _sc_mesh_kernel():
  mesh = plsc.VectorSubcoreMesh(core_axis_name="c", subcore_axis_name="s",
                                num_cores=NC, num_subcores=NS)

  @functools.partial(
      pl.kernel,
      out_type=(
          jax.ShapeDtypeStruct((N,), jnp.float32),
          jax.ShapeDtypeStruct((E,), jnp.float32),
          jax.ShapeDtypeStruct((E,), jnp.float32),
          jax.ShapeDtypeStruct((E,), jnp.float32),
      ),
      mesh=mesh,
      compiler_params=pltpu.CompilerParams(needs_layout_passes=False),
      scratch_types=[
          pltpu.VMEM((N,), jnp.float32),
          pltpu.VMEM((N,), jnp.float32),
          pltpu.VMEM((N,), jnp.float32),
          pltpu.VMEM((EPW,), jnp.int32),
          pltpu.VMEM((EPW,), jnp.int32),
          pltpu.VMEM((EPW,), jnp.float32),
          pltpu.VMEM((EPW,), jnp.float32),
          pltpu.VMEM((EPW,), jnp.float32),
          pltpu.VMEM((_ROWS_PER_TILE, 128), jnp.int32),
          pltpu.VMEM((128,), jnp.float32),
          pltpu.VMEM((2000,), jnp.float32),
          pltpu.VMEM_SHARED((N,), jnp.float32),
          pltpu.SemaphoreType.DMA,
          pltpu.SemaphoreType.DMA,
      ],
  )
  def k(x_hbm, y_hbm, z_hbm, src_hbm, dst_hbm, dst2d_hbm,
        deg_out, dx_out, dy_out, dz_out,
        xv, yv, zv, sv, dv, dxv, dyv, dzv, degidx, onesv, zerov, degsp,
        sem_in, sem_idx):
    c = lax.axis_index("c")
    s = lax.axis_index("s")
    wid = s * NC + c
    base = wid * EPW

    pltpu.async_copy(x_hbm, xv, sem_in)
    pltpu.async_copy(y_hbm, yv, sem_in)
    pltpu.async_copy(z_hbm, zv, sem_in)
    pltpu.async_copy(src_hbm.at[pl.ds(base, EPW)], sv, sem_in)
    pltpu.async_copy(dst_hbm.at[pl.ds(base, EPW)], dv, sem_in)

    nbase = s * _ROWS_PER_TILE
    nrows = jnp.minimum(_ROWS_PER_TILE, DEG_ROWS - nbase)

    @pl.when(c == 0)
    def _():
      pltpu.async_copy(dst2d_hbm.at[pl.ds(nbase, _ROWS_PER_TILE)], degidx,
                       sem_idx)

    zf = jnp.zeros((L,), jnp.float32)

    def zstep(i, _):
      zerov[pl.ds(i * L, L)] = zf
      return 0
    lax.fori_loop(0, 2000 // L, zstep, 0)
    onesv[pl.ds(0, L)] = jnp.ones((L,), jnp.float32)
    for j in range(1, 128 // L):
      onesv[pl.ds(j * L, L)] = jnp.ones((L,), jnp.float32)

    @pl.when(s < 5)
    def _():
      pltpu.sync_copy(zerov, degsp.at[pl.ds(s * 2000, 2000)])

    plsc.subcore_barrier()

    @pl.when(c == 0)
    def _():
      pltpu.make_async_copy(
          dst2d_hbm.at[pl.ds(nbase, _ROWS_PER_TILE)], degidx, sem_idx).wait()

      def dstep(r, _):
        pltpu.sync_copy(onesv, degsp.at[degidx.at[r]], add=True)
        return 0
      lax.fori_loop(0, nrows, dstep, 0)

    pltpu.make_async_copy(x_hbm, xv, sem_in).wait()
    pltpu.make_async_copy(y_hbm, yv, sem_in).wait()
    pltpu.make_async_copy(z_hbm, zv, sem_in).wait()
    pltpu.make_async_copy(src_hbm.at[pl.ds(base, EPW)], sv, sem_in).wait()
    pltpu.make_async_copy(dst_hbm.at[pl.ds(base, EPW)], dv, sem_in).wait()

    @plsc.parallel_loop(0, EPW // L, unroll=8)
    def dir_step(i):
      off = i * L
      sidx = sv[pl.ds(off, L)]
      didx = dv[pl.ds(off, L)]
      xs = plsc.load_gather(xv, [sidx])
      ys = plsc.load_gather(yv, [sidx])
      zs = plsc.load_gather(zv, [sidx])
      xd = plsc.load_gather(xv, [didx])
      yd = plsc.load_gather(yv, [didx])
      zd = plsc.load_gather(zv, [didx])
      dx = xd - xs
      dy = yd - ys
      dz = zd - zs
      n2 = dx * dx + dy * dy + dz * dz
      bits = plsc.bitcast(n2, jnp.int32)
      yf = plsc.bitcast(jnp.int32(0x5F3759DF) - (bits >> 1), jnp.float32)
      yf = yf * (1.5 - 0.5 * n2 * yf * yf)
      yf = yf * (1.5 - 0.5 * n2 * yf * yf)
      yf = yf * (1.5 - 0.5 * n2 * yf * yf)
      norm = n2 * yf
      inv = 1.0 / (norm + 1e-8)
      dxv[pl.ds(off, L)] = dx * inv
      dyv[pl.ds(off, L)] = dy * inv
      dzv[pl.ds(off, L)] = dz * inv

    pltpu.async_copy(dxv, dx_out.at[pl.ds(base, EPW)], sem_in)
    pltpu.async_copy(dyv, dy_out.at[pl.ds(base, EPW)], sem_in)
    pltpu.async_copy(dzv, dz_out.at[pl.ds(base, EPW)], sem_in)

    pltpu.make_async_copy(dxv, dx_out.at[pl.ds(base, EPW)], sem_in).wait()
    pltpu.make_async_copy(dyv, dy_out.at[pl.ds(base, EPW)], sem_in).wait()
    pltpu.make_async_copy(dzv, dz_out.at[pl.ds(base, EPW)], sem_in).wait()

    plsc.subcore_barrier()

    @pl.when((c == 0) & (s < 10))
    def _():
      stage = zerov.at[pl.ds(0, 1000)]
      pltpu.sync_copy(degsp.at[pl.ds(s * 1000, 1000)], stage)
      pltpu.sync_copy(stage, deg_out.at[pl.ds(s * 1000, 1000)])

  return k


def _tc_edges(dist_bins, bond_types, bd_table, block_e=2048):
  grid = (pl.cdiv(E, block_e),)

  def body(dis_ref, bond_ref, bd_ref, dis_out, bond_out):
    dis = dis_ref[...].reshape(1, block_e)
    bond = bond_ref[...].reshape(1, block_e)
    rows = lax.broadcasted_iota(jnp.int32, (128, block_e), 0)
    oht = jnp.where((rows == dis) | (rows == bond + NUM_DIST),
                    jnp.float32(1.0), jnp.float32(0.0))
    r = lax.dot_general(oht, bd_ref[...],
                        dimension_numbers=(((0,), (0,)), ((), ())),
                        preferred_element_type=jnp.float32)
    dis_out[...] = r[:, :D]
    bond_out[...] = r[:, D:]

  return pl.pallas_call(
      body,
      grid=grid,
      in_specs=[
          pl.BlockSpec((block_e,), lambda i: (i,)),
          pl.BlockSpec((block_e,), lambda i: (i,)),
          pl.BlockSpec((128, 2 * D), lambda i: (0, 0)),
      ],
      out_specs=[
          pl.BlockSpec((block_e, D), lambda i: (i, 0)),
          pl.BlockSpec((block_e, D), lambda i: (i, 0)),
      ],
      out_shape=[
          jax.ShapeDtypeStruct((E, D), jnp.float32),
          jax.ShapeDtypeStruct((E, D), jnp.float32),
      ],
  )(dist_bins, bond_types, bd_table)


def _tc_nodes(atom_types, atom_pad, block_n=2048):
  grid = (pl.cdiv(N, block_n),)

  def body(idx_ref, tab_ref, out_ref):
    idx = idx_ref[...].reshape(1, block_n)
    rows = lax.broadcasted_iota(jnp.int32, (128, block_n), 0)
    oht = jnp.where(rows == idx, jnp.float32(1.0), jnp.float32(0.0))
    out_ref[...] = lax.dot_general(
        oht, tab_ref[...],
        dimension_numbers=(((0,), (0,)), ((), ())),
        preferred_element_type=jnp.float32)

  return pl.pallas_call(
      body,
      grid=grid,
      in_specs=[
          pl.BlockSpec((block_n,), lambda i: (i,)),
          pl.BlockSpec((128, D), lambda i: (0, 0)),
      ],
      out_specs=pl.BlockSpec((block_n, D), lambda i: (i, 0)),
      out_shape=jax.ShapeDtypeStruct((N, D), jnp.float32),
  )(atom_types, atom_pad)


def kernel(atom_types, edge_index, bond_types, dist_bins, pos,
           atom_table, bond_table, dist_table):
  edge_index = edge_index.astype(jnp.int32)
  atom_types = atom_types.astype(jnp.int32)
  bond_types = bond_types.astype(jnp.int32)
  dist_bins = dist_bins.astype(jnp.int32)

  pos_x = pos[:, 0]
  pos_y = pos[:, 1]
  pos_z = pos[:, 2]
  src = edge_index[0]
  dst = edge_index[1]
  dst_pad = jnp.concatenate(
      [edge_index[1], jnp.zeros((DST2D_ROWS * 128 - E,), jnp.int32)])
  dst2d = dst_pad.reshape(DST2D_ROWS, 128)

  bd = jnp.zeros((128, 2 * D), jnp.float32)
  bd = bd.at[:NUM_DIST, :D].set(dist_table)
  bd = bd.at[NUM_DIST:NUM_DIST + NUM_BOND, D:].set(bond_table)
  atom_pad = jnp.zeros((128, D), jnp.float32).at[:NUM_ATOM].set(atom_table)

  degree, dir_x, dir_y, dir_z = _sc_mesh_kernel()(
      pos_x, pos_y, pos_z, src, dst, dst2d)
  edges_direction = jnp.stack([dir_x, dir_y, dir_z], axis=1)
  edge_feat_dis, edge_feat_bond = _tc_edges(dist_bins, bond_types, bd)
  node_feat = _tc_nodes(atom_types, atom_pad)

  return (node_feat, edge_feat_dis, edge_feat_bond, degree, edges_direction)


# speedup vs baseline: 1.2481x; 1.2481x over previous
import functools

import jax
import jax.numpy as jnp
from jax import lax
from jax.experimental import pallas as pl
from jax.experimental.pallas import tpu as pltpu
from jax.experimental.pallas import tpu_sc as plsc

N = 10000
E = 320000
D = 128
NUM_ATOM = 119
NUM_BOND = 8
NUM_DIST = 64

NC = 2
NS = 16
L = 16
NW = NC * NS

EPW = E // NW
DEG_ROWS = E // 128
_ROWS_PER_TILE = 160
DST2D_ROWS = _ROWS_PER_TILE * NS


@functools.cache
def _sc_mesh_kernel():
  mesh = plsc.VectorSubcoreMesh(core_axis_name="c", subcore_axis_name="s",
                                num_cores=NC, num_subcores=NS)

  @functools.partial(
      pl.kernel,
      out_type=(
          jax.ShapeDtypeStruct((N,), jnp.float32),
          jax.ShapeDtypeStruct((E,), jnp.float32),
          jax.ShapeDtypeStruct((E,), jnp.float32),
          jax.ShapeDtypeStruct((E,), jnp.float32),
      ),
      mesh=mesh,
      compiler_params=pltpu.CompilerParams(needs_layout_passes=False),
      scratch_types=[
          pltpu.VMEM((N,), jnp.float32),
          pltpu.VMEM((N,), jnp.float32),
          pltpu.VMEM((N,), jnp.float32),
          pltpu.VMEM((EPW,), jnp.int32),
          pltpu.VMEM((EPW,), jnp.int32),
          pltpu.VMEM((EPW,), jnp.float32),
          pltpu.VMEM((EPW,), jnp.float32),
          pltpu.VMEM((EPW,), jnp.float32),
          pltpu.VMEM((_ROWS_PER_TILE, 128), jnp.int32),
          pltpu.VMEM((128,), jnp.float32),
          pltpu.VMEM((2000,), jnp.float32),
          pltpu.VMEM_SHARED((N,), jnp.float32),
          pltpu.SemaphoreType.DMA,
          pltpu.SemaphoreType.DMA,
      ],
  )
  def k(x_hbm, y_hbm, z_hbm, src_hbm, dst_hbm, dst2d_hbm,
        deg_out, dx_out, dy_out, dz_out,
        xv, yv, zv, sv, dv, dxv, dyv, dzv, degidx, onesv, zerov, degsp,
        sem_in, sem_idx):
    c = lax.axis_index("c")
    s = lax.axis_index("s")
    wid = s * NC + c
    base = wid * EPW

    pltpu.async_copy(x_hbm, xv, sem_in)
    pltpu.async_copy(y_hbm, yv, sem_in)
    pltpu.async_copy(z_hbm, zv, sem_in)
    pltpu.async_copy(src_hbm.at[pl.ds(base, EPW)], sv, sem_in)
    pltpu.async_copy(dst_hbm.at[pl.ds(base, EPW)], dv, sem_in)

    nbase = s * _ROWS_PER_TILE
    nrows = jnp.minimum(_ROWS_PER_TILE, DEG_ROWS - nbase)

    @pl.when(c == 0)
    def _():
      pltpu.async_copy(dst2d_hbm.at[pl.ds(nbase, _ROWS_PER_TILE)], degidx,
                       sem_idx)

    zf = jnp.zeros((L,), jnp.float32)

    def zstep(i, _):
      zerov[pl.ds(i * L, L)] = zf
      return 0
    lax.fori_loop(0, 2000 // L, zstep, 0)
    onesv[pl.ds(0, L)] = jnp.ones((L,), jnp.float32)
    for j in range(1, 128 // L):
      onesv[pl.ds(j * L, L)] = jnp.ones((L,), jnp.float32)

    @pl.when(s < 5)
    def _():
      pltpu.sync_copy(zerov, degsp.at[pl.ds(s * 2000, 2000)])

    plsc.subcore_barrier()

    @pl.when(c == 0)
    def _():
      pltpu.make_async_copy(
          dst2d_hbm.at[pl.ds(nbase, _ROWS_PER_TILE)], degidx, sem_idx).wait()

      def dstep(r, _):
        pltpu.sync_copy(onesv, degsp.at[degidx.at[r]], add=True)
        return 0
      lax.fori_loop(0, nrows, dstep, 0)

    pltpu.make_async_copy(x_hbm, xv, sem_in).wait()
    pltpu.make_async_copy(y_hbm, yv, sem_in).wait()
    pltpu.make_async_copy(z_hbm, zv, sem_in).wait()
    pltpu.make_async_copy(src_hbm.at[pl.ds(base, EPW)], sv, sem_in).wait()
    pltpu.make_async_copy(dst_hbm.at[pl.ds(base, EPW)], dv, sem_in).wait()

    @plsc.parallel_loop(0, EPW // L, unroll=8)
    def dir_step(i):
      off = i * L
      sidx = sv[pl.ds(off, L)]
      didx = dv[pl.ds(off, L)]
      xs = plsc.load_gather(xv, [sidx])
      ys = plsc.load_gather(yv, [sidx])
      zs = plsc.load_gather(zv, [sidx])
      xd = plsc.load_gather(xv, [didx])
      yd = plsc.load_gather(yv, [didx])
      zd = plsc.load_gather(zv, [didx])
      dx = xd - xs
      dy = yd - ys
      dz = zd - zs
      n2 = dx * dx + dy * dy + dz * dz
      bits = plsc.bitcast(n2, jnp.int32)
      yf = plsc.bitcast(jnp.int32(0x5F3759DF) - (bits >> 1), jnp.float32)
      yf = yf * (1.5 - 0.5 * n2 * yf * yf)
      yf = yf * (1.5 - 0.5 * n2 * yf * yf)
      yf = yf * (1.5 - 0.5 * n2 * yf * yf)
      norm = n2 * yf
      inv = 1.0 / (norm + 1e-8)
      dxv[pl.ds(off, L)] = dx * inv
      dyv[pl.ds(off, L)] = dy * inv
      dzv[pl.ds(off, L)] = dz * inv

    pltpu.async_copy(dxv, dx_out.at[pl.ds(base, EPW)], sem_in)
    pltpu.async_copy(dyv, dy_out.at[pl.ds(base, EPW)], sem_in)
    pltpu.async_copy(dzv, dz_out.at[pl.ds(base, EPW)], sem_in)

    pltpu.make_async_copy(dxv, dx_out.at[pl.ds(base, EPW)], sem_in).wait()
    pltpu.make_async_copy(dyv, dy_out.at[pl.ds(base, EPW)], sem_in).wait()
    pltpu.make_async_copy(dzv, dz_out.at[pl.ds(base, EPW)], sem_in).wait()

    plsc.subcore_barrier()

    @pl.when((c == 0) & (s < 10))
    def _():
      stage = zerov.at[pl.ds(0, 1000)]
      pltpu.sync_copy(degsp.at[pl.ds(s * 1000, 1000)], stage)
      pltpu.sync_copy(stage, deg_out.at[pl.ds(s * 1000, 1000)])

  return k


def _tc_edges(dist_bins, bond_types, bd_table, block_e=4096):
  grid = (pl.cdiv(E, block_e),)

  def body(dis_ref, bond_ref, bd_ref, dis_out, bond_out):
    dis = dis_ref[...].reshape(1, block_e)
    bond = bond_ref[...].reshape(1, block_e)
    rows = lax.broadcasted_iota(jnp.int32, (128, block_e), 0)
    oht = jnp.where((rows == dis) | (rows == bond + NUM_DIST),
                    jnp.float32(1.0), jnp.float32(0.0))
    r = lax.dot_general(oht, bd_ref[...],
                        dimension_numbers=(((0,), (0,)), ((), ())),
                        preferred_element_type=jnp.float32)
    dis_out[...] = r[:, :D]
    bond_out[...] = r[:, D:]

  return pl.pallas_call(
      body,
      grid=grid,
      in_specs=[
          pl.BlockSpec((block_e,), lambda i: (i,)),
          pl.BlockSpec((block_e,), lambda i: (i,)),
          pl.BlockSpec((128, 2 * D), lambda i: (0, 0)),
      ],
      out_specs=[
          pl.BlockSpec((block_e, D), lambda i: (i, 0)),
          pl.BlockSpec((block_e, D), lambda i: (i, 0)),
      ],
      out_shape=[
          jax.ShapeDtypeStruct((E, D), jnp.float32),
          jax.ShapeDtypeStruct((E, D), jnp.float32),
      ],
  )(dist_bins, bond_types, bd_table)


def _tc_nodes(atom_types, atom_pad, block_n=2048):
  grid = (pl.cdiv(N, block_n),)

  def body(idx_ref, tab_ref, out_ref):
    idx = idx_ref[...].reshape(1, block_n)
    rows = lax.broadcasted_iota(jnp.int32, (128, block_n), 0)
    oht = jnp.where(rows == idx, jnp.float32(1.0), jnp.float32(0.0))
    out_ref[...] = lax.dot_general(
        oht, tab_ref[...],
        dimension_numbers=(((0,), (0,)), ((), ())),
        preferred_element_type=jnp.float32)

  return pl.pallas_call(
      body,
      grid=grid,
      in_specs=[
          pl.BlockSpec((block_n,), lambda i: (i,)),
          pl.BlockSpec((128, D), lambda i: (0, 0)),
      ],
      out_specs=pl.BlockSpec((block_n, D), lambda i: (i, 0)),
      out_shape=jax.ShapeDtypeStruct((N, D), jnp.float32),
  )(atom_types, atom_pad)


def kernel(atom_types, edge_index, bond_types, dist_bins, pos,
           atom_table, bond_table, dist_table):
  edge_index = edge_index.astype(jnp.int32)
  atom_types = atom_types.astype(jnp.int32)
  bond_types = bond_types.astype(jnp.int32)
  dist_bins = dist_bins.astype(jnp.int32)

  pos_x = pos[:, 0]
  pos_y = pos[:, 1]
  pos_z = pos[:, 2]
  src = edge_index[0]
  dst = edge_index[1]
  dst_pad = jnp.concatenate(
      [edge_index[1], jnp.zeros((DST2D_ROWS * 128 - E,), jnp.int32)])
  dst2d = dst_pad.reshape(DST2D_ROWS, 128)

  bd = jnp.zeros((128, 2 * D), jnp.float32)
  bd = bd.at[:NUM_DIST, :D].set(dist_table)
  bd = bd.at[NUM_DIST:NUM_DIST + NUM_BOND, D:].set(bond_table)
  atom_pad = jnp.zeros((128, D), jnp.float32).at[:NUM_ATOM].set(atom_table)

  degree, dir_x, dir_y, dir_z = _sc_mesh_kernel()(
      pos_x, pos_y, pos_z, src, dst, dst2d)
  edges_direction = jnp.stack([dir_x, dir_y, dir_z], axis=1)
  edge_feat_dis, edge_feat_bond = _tc_edges(dist_bins, bond_types, bd)
  node_feat = _tc_nodes(atom_types, atom_pad)

  return (node_feat, edge_feat_dis, edge_feat_bond, degree, edges_direction)


# speedup vs baseline: 1.3326x; 1.0677x over previous
import functools

import jax
import jax.numpy as jnp
from jax import lax
from jax.experimental import pallas as pl
from jax.experimental.pallas import tpu as pltpu
from jax.experimental.pallas import tpu_sc as plsc

N = 10000
E = 320000
D = 128
NUM_ATOM = 119
NUM_BOND = 8
NUM_DIST = 64

NC = 2
NS = 16
L = 16
NW = NC * NS

EPW = E // NW
DEG_ROWS = E // 128
_ROWS_PER_TILE = 160
DST2D_ROWS = _ROWS_PER_TILE * NS


@functools.cache
def _sc_mesh_kernel():
  mesh = plsc.VectorSubcoreMesh(core_axis_name="c", subcore_axis_name="s",
                                num_cores=NC, num_subcores=NS)

  @functools.partial(
      pl.kernel,
      out_type=(
          jax.ShapeDtypeStruct((N,), jnp.float32),
          jax.ShapeDtypeStruct((E,), jnp.float32),
          jax.ShapeDtypeStruct((E,), jnp.float32),
          jax.ShapeDtypeStruct((E,), jnp.float32),
      ),
      mesh=mesh,
      compiler_params=pltpu.CompilerParams(needs_layout_passes=False),
      scratch_types=[
          pltpu.VMEM((N,), jnp.float32),
          pltpu.VMEM((N,), jnp.float32),
          pltpu.VMEM((N,), jnp.float32),
          pltpu.VMEM((EPW,), jnp.int32),
          pltpu.VMEM((EPW,), jnp.int32),
          pltpu.VMEM((EPW,), jnp.float32),
          pltpu.VMEM((EPW,), jnp.float32),
          pltpu.VMEM((EPW,), jnp.float32),
          pltpu.VMEM((_ROWS_PER_TILE, 128), jnp.int32),
          pltpu.VMEM((128,), jnp.float32),
          pltpu.VMEM((2000,), jnp.float32),
          pltpu.VMEM_SHARED((N,), jnp.float32),
          pltpu.SemaphoreType.DMA,
          pltpu.SemaphoreType.DMA,
      ],
  )
  def k(x_hbm, y_hbm, z_hbm, src_hbm, dst_hbm, dst2d_hbm,
        deg_out, dx_out, dy_out, dz_out,
        xv, yv, zv, sv, dv, dxv, dyv, dzv, degidx, onesv, zerov, degsp,
        sem_in, sem_idx):
    c = lax.axis_index("c")
    s = lax.axis_index("s")
    wid = s * NC + c
    base = wid * EPW

    pltpu.async_copy(x_hbm, xv, sem_in)
    pltpu.async_copy(y_hbm, yv, sem_in)
    pltpu.async_copy(z_hbm, zv, sem_in)
    pltpu.async_copy(src_hbm.at[pl.ds(base, EPW)], sv, sem_in)
    pltpu.async_copy(dst_hbm.at[pl.ds(base, EPW)], dv, sem_in)

    nbase = s * _ROWS_PER_TILE
    nrows = jnp.minimum(_ROWS_PER_TILE, DEG_ROWS - nbase)

    @pl.when(c == 0)
    def _():
      pltpu.async_copy(dst2d_hbm.at[pl.ds(nbase, _ROWS_PER_TILE)], degidx,
                       sem_idx)

    zf = jnp.zeros((L,), jnp.float32)

    def zstep(i, _):
      zerov[pl.ds(i * L, L)] = zf
      return 0
    lax.fori_loop(0, 2000 // L, zstep, 0)
    onesv[pl.ds(0, L)] = jnp.ones((L,), jnp.float32)
    for j in range(1, 128 // L):
      onesv[pl.ds(j * L, L)] = jnp.ones((L,), jnp.float32)

    @pl.when(s < 5)
    def _():
      pltpu.sync_copy(zerov, degsp.at[pl.ds(s * 2000, 2000)])

    plsc.subcore_barrier()

    @pl.when(c == 0)
    def _():
      pltpu.make_async_copy(
          dst2d_hbm.at[pl.ds(nbase, _ROWS_PER_TILE)], degidx, sem_idx).wait()

      def dstep(r, _):
        pltpu.sync_copy(onesv, degsp.at[degidx.at[r]], add=True)
        return 0
      lax.fori_loop(0, nrows, dstep, 0)

    pltpu.make_async_copy(x_hbm, xv, sem_in).wait()
    pltpu.make_async_copy(y_hbm, yv, sem_in).wait()
    pltpu.make_async_copy(z_hbm, zv, sem_in).wait()
    pltpu.make_async_copy(src_hbm.at[pl.ds(base, EPW)], sv, sem_in).wait()
    pltpu.make_async_copy(dst_hbm.at[pl.ds(base, EPW)], dv, sem_in).wait()

    @plsc.parallel_loop(0, EPW // L, unroll=8)
    def dir_step(i):
      off = i * L
      sidx = sv[pl.ds(off, L)]
      didx = dv[pl.ds(off, L)]
      xs = plsc.load_gather(xv, [sidx])
      ys = plsc.load_gather(yv, [sidx])
      zs = plsc.load_gather(zv, [sidx])
      xd = plsc.load_gather(xv, [didx])
      yd = plsc.load_gather(yv, [didx])
      zd = plsc.load_gather(zv, [didx])
      dx = xd - xs
      dy = yd - ys
      dz = zd - zs
      n2 = dx * dx + dy * dy + dz * dz
      bits = plsc.bitcast(n2, jnp.int32)
      yf = plsc.bitcast(jnp.int32(0x5F3759DF) - (bits >> 1), jnp.float32)
      yf = yf * (1.5 - 0.5 * n2 * yf * yf)
      yf = yf * (1.5 - 0.5 * n2 * yf * yf)
      yf = yf * (1.5 - 0.5 * n2 * yf * yf)
      norm = n2 * yf
      inv = 1.0 / (norm + 1e-8)
      dxv[pl.ds(off, L)] = dx * inv
      dyv[pl.ds(off, L)] = dy * inv
      dzv[pl.ds(off, L)] = dz * inv

    pltpu.async_copy(dxv, dx_out.at[pl.ds(base, EPW)], sem_in)
    pltpu.async_copy(dyv, dy_out.at[pl.ds(base, EPW)], sem_in)
    pltpu.async_copy(dzv, dz_out.at[pl.ds(base, EPW)], sem_in)

    pltpu.make_async_copy(dxv, dx_out.at[pl.ds(base, EPW)], sem_in).wait()
    pltpu.make_async_copy(dyv, dy_out.at[pl.ds(base, EPW)], sem_in).wait()
    pltpu.make_async_copy(dzv, dz_out.at[pl.ds(base, EPW)], sem_in).wait()

    plsc.subcore_barrier()

    @pl.when((c == 0) & (s < 10))
    def _():
      stage = zerov.at[pl.ds(0, 1000)]
      pltpu.sync_copy(degsp.at[pl.ds(s * 1000, 1000)], stage)
      pltpu.sync_copy(stage, deg_out.at[pl.ds(s * 1000, 1000)])

  return k


def _tc_edges(dist_bins, bond_types, bd_table, block_e=8192):
  grid = (pl.cdiv(E, block_e),)

  def body(dis_ref, bond_ref, bd_ref, dis_out, bond_out):
    dis = dis_ref[...].reshape(1, block_e)
    bond = bond_ref[...].reshape(1, block_e)
    rows = lax.broadcasted_iota(jnp.int32, (128, block_e), 0)
    oht = jnp.where((rows == dis) | (rows == bond + NUM_DIST),
                    jnp.float32(1.0), jnp.float32(0.0))
    r = lax.dot_general(oht, bd_ref[...],
                        dimension_numbers=(((0,), (0,)), ((), ())),
                        preferred_element_type=jnp.float32)
    dis_out[...] = r[:, :D]
    bond_out[...] = r[:, D:]

  return pl.pallas_call(
      body,
      grid=grid,
      in_specs=[
          pl.BlockSpec((block_e,), lambda i: (i,)),
          pl.BlockSpec((block_e,), lambda i: (i,)),
          pl.BlockSpec((128, 2 * D), lambda i: (0, 0)),
      ],
      out_specs=[
          pl.BlockSpec((block_e, D), lambda i: (i, 0)),
          pl.BlockSpec((block_e, D), lambda i: (i, 0)),
      ],
      out_shape=[
          jax.ShapeDtypeStruct((E, D), jnp.float32),
          jax.ShapeDtypeStruct((E, D), jnp.float32),
      ],
  )(dist_bins, bond_types, bd_table)


def _tc_nodes(atom_types, atom_pad, block_n=2048):
  grid = (pl.cdiv(N, block_n),)

  def body(idx_ref, tab_ref, out_ref):
    idx = idx_ref[...].reshape(1, block_n)
    rows = lax.broadcasted_iota(jnp.int32, (128, block_n), 0)
    oht = jnp.where(rows == idx, jnp.float32(1.0), jnp.float32(0.0))
    out_ref[...] = lax.dot_general(
        oht, tab_ref[...],
        dimension_numbers=(((0,), (0,)), ((), ())),
        preferred_element_type=jnp.float32)

  return pl.pallas_call(
      body,
      grid=grid,
      in_specs=[
          pl.BlockSpec((block_n,), lambda i: (i,)),
          pl.BlockSpec((128, D), lambda i: (0, 0)),
      ],
      out_specs=pl.BlockSpec((block_n, D), lambda i: (i, 0)),
      out_shape=jax.ShapeDtypeStruct((N, D), jnp.float32),
  )(atom_types, atom_pad)


def kernel(atom_types, edge_index, bond_types, dist_bins, pos,
           atom_table, bond_table, dist_table):
  edge_index = edge_index.astype(jnp.int32)
  atom_types = atom_types.astype(jnp.int32)
  bond_types = bond_types.astype(jnp.int32)
  dist_bins = dist_bins.astype(jnp.int32)

  pos_x = pos[:, 0]
  pos_y = pos[:, 1]
  pos_z = pos[:, 2]
  src = edge_index[0]
  dst = edge_index[1]
  dst_pad = jnp.concatenate(
      [edge_index[1], jnp.zeros((DST2D_ROWS * 128 - E,), jnp.int32)])
  dst2d = dst_pad.reshape(DST2D_ROWS, 128)

  bd = jnp.zeros((128, 2 * D), jnp.float32)
  bd = bd.at[:NUM_DIST, :D].set(dist_table)
  bd = bd.at[NUM_DIST:NUM_DIST + NUM_BOND, D:].set(bond_table)
  atom_pad = jnp.zeros((128, D), jnp.float32).at[:NUM_ATOM].set(atom_table)

  degree, dir_x, dir_y, dir_z = _sc_mesh_kernel()(
      pos_x, pos_y, pos_z, src, dst, dst2d)
  edges_direction = jnp.stack([dir_x, dir_y, dir_z], axis=1)
  edge_feat_dis, edge_feat_bond = _tc_edges(dist_bins, bond_types, bd)
  node_feat = _tc_nodes(atom_types, atom_pad)

  return (node_feat, edge_feat_dis, edge_feat_bond, degree, edges_direction)
